# baseline (speedup 1.0000x reference)
import jax
import jax.numpy as jnp
from jax import lax
from jax.experimental import pallas as pl
from jax.experimental.pallas import tpu as pltpu
from jax.experimental.pallas import tpu_sc as plsc

N_NODES = 10000
D_FEAT = 128
HIDDEN = 256
N_EDGES = 320000

CHUNK = 128
NSUB = 16
NCORE = 2
NSPAN = NCORE * NSUB
CPT = 80
IBLK = 8
E_PAD = NSPAN * CPT * CHUNK
AGG_ROWS = N_NODES + 8
RMAIN = 624
RTAIL = N_NODES - NSUB * RMAIN

BM = 1000


def _run_spans(x_rows_hbm, srcm_hbm, dstm_hbm, agg, sbuf, dbuf, rows,
               gsem0, gsem1, ssem0, ssem1, span_ids):
  nsp = len(span_ids)
  T = nsp * CPT
  nblk = T // IBLK

  def span_chunk(g):
    if nsp == 1:
      return span_ids[0], g
    sp = g // CPT
    return jnp.where(sp == 0, span_ids[0], span_ids[1]), g - sp * CPT

  def stage(kb):
    w, gg = span_chunk(kb * IBLK)
    o = pl.multiple_of(gg, IBLK)
    d = pl.ds(pl.multiple_of(lax.rem(kb, 2) * IBLK, IBLK), IBLK)
    pltpu.sync_copy(srcm_hbm.at[w].at[pl.ds(o, IBLK)], sbuf.at[d])
    pltpu.sync_copy(dstm_hbm.at[w].at[pl.ds(o, IBLK)], dbuf.at[d])

  def gather(g, rbuf, sem):
    return pltpu.make_async_copy(
        x_rows_hbm.at[sbuf.at[lax.rem(g, 2 * IBLK)]], rbuf, sem)

  def scatter_start(g, rbuf, sem):
    pltpu.async_copy(rbuf, agg.at[dbuf.at[lax.rem(g, 2 * IBLK)]], sem,
                     add=True)

  def scatter_wait(g, rbuf, sem):
    pltpu.make_async_copy(rbuf, agg.at[dbuf.at[lax.rem(g, 2 * IBLK)]],
                          sem).wait()

  stage(0)
  gather(0, rows.at[0], gsem0).start()
  gather(1, rows.at[1], gsem1).start()

  def body(t, carry):
    g0 = 2 * t
    g1 = g0 + 1

    @pl.when(lax.rem(g0, IBLK) == 0)
    def _():
      kb = g0 // IBLK

      @pl.when(kb + 1 < nblk)
      def _():
        stage(kb + 1)

    gather(g0, rows.at[0], gsem0).wait()
    scatter_start(g0, rows.at[0], ssem0)
    gather(g1, rows.at[1], gsem1).wait()
    scatter_start(g1, rows.at[1], ssem1)

    scatter_wait(g0, rows.at[0], ssem0)

    @pl.when(g0 + 2 < T)
    def _():
      gather(g0 + 2, rows.at[0], gsem0).start()

    scatter_wait(g1, rows.at[1], ssem1)

    @pl.when(g1 + 2 < T)
    def _():
      gather(g1 + 2, rows.at[1], gsem1).start()

    return carry

  lax.fori_loop(0, T // 2, body, 0)


def _copy_node_rows(src_get, dst_put, s):
  r0 = s * RMAIN
  dst_put(pl.ds(r0, RMAIN), src_get(pl.ds(r0, RMAIN)))

  @pl.when(s == 0)
  def _():
    dst_put(pl.ds(NSUB * RMAIN, RTAIL), src_get(pl.ds(NSUB * RMAIN, RTAIL)))


def _agg1_body(x_hbm, z_hbm, srcm_hbm, dstm_hbm, out_hbm,
               agg, sbuf, dbuf, rows, gsem0, gsem1, ssem0, ssem1):
  c = lax.axis_index("c")
  s = lax.axis_index("s")

  @pl.when(c == 0)
  def _():
    _copy_node_rows(lambda d: x_hbm.at[d], lambda d, r: pltpu.sync_copy(r, agg.at[d]), s)

  @pl.when(c != 0)
  def _():
    pltpu.sync_copy(z_hbm, agg.at[pl.ds(s * RMAIN, RMAIN)])

    @pl.when(s == 0)
    def _():
      pltpu.sync_copy(z_hbm.at[pl.ds(0, RTAIL)],
                      agg.at[pl.ds(NSUB * RMAIN, RTAIL)])

  plsc.subcore_barrier()
  _run_spans(x_hbm, srcm_hbm, dstm_hbm, agg, sbuf, dbuf, rows,
             gsem0, gsem1, ssem0, ssem1, [c * NSUB + s])
  plsc.subcore_barrier()
  _copy_node_rows(lambda d: agg.at[d],
                  lambda d, r: pltpu.sync_copy(r, out_hbm.at[c].at[d]), s)


def _agg23_body(xs_hbm, srcm_hbm, dstm_hbm, out_hbm,
                agg, sbuf, dbuf, rows, gsem0, gsem1, ssem0, ssem1):
  c = lax.axis_index("c")
  s = lax.axis_index("s")
  xc = xs_hbm.at[c]
  _copy_node_rows(lambda d: xc.at[d], lambda d, r: pltpu.sync_copy(r, agg.at[d]), s)
  plsc.subcore_barrier()
  _run_spans(xc, srcm_hbm, dstm_hbm, agg, sbuf, dbuf, rows,
             gsem0, gsem1, ssem0, ssem1, [2 * s, 2 * s + 1])
  plsc.subcore_barrier()
  _copy_node_rows(lambda d: agg.at[d],
                  lambda d, r: pltpu.sync_copy(r, out_hbm.at[c].at[d]), s)


def _make_agg(body):
  mesh = plsc.VectorSubcoreMesh(core_axis_name="c", subcore_axis_name="s")
  return pl.kernel(
      body,
      out_type=jax.ShapeDtypeStruct((NCORE, N_NODES, D_FEAT), jnp.float32),
      mesh=mesh,
      scratch_types=[
          pltpu.VMEM_SHARED((AGG_ROWS, D_FEAT), jnp.float32),
          pltpu.VMEM((2 * IBLK, CHUNK), jnp.int32),
          pltpu.VMEM((2 * IBLK, CHUNK), jnp.int32),
          pltpu.VMEM((2, CHUNK, D_FEAT), jnp.float32),
          pltpu.SemaphoreType.DMA,
          pltpu.SemaphoreType.DMA,
          pltpu.SemaphoreType.DMA,
          pltpu.SemaphoreType.DMA,
      ],
  )


_agg1 = _make_agg(_agg1_body)
_agg23 = _make_agg(_agg23_body)


def _relu(v):
  return jnp.maximum(v, 0.0)


def _mlp1_body(p_ref, wa_ref, ba_ref, wb_ref, bb_ref, out_ref):
  h0 = p_ref[0] + p_ref[1]
  h1 = _relu(jnp.dot(h0, wa_ref[...], preferred_element_type=jnp.float32)
             + ba_ref[...])
  h2 = _relu(jnp.dot(h1, wb_ref[...], preferred_element_type=jnp.float32)
             + bb_ref[...])
  out_ref[0] = h2[:, :D_FEAT]
  out_ref[1] = h2[:, D_FEAT:]


def _mlp2_body(p_ref, r_ref, wa_ref, ba_ref, wb_ref, bb_ref, out_ref):
  h0 = jnp.concatenate([p_ref[0], p_ref[1]], axis=1)
  h1 = _relu(jnp.dot(h0, wa_ref[...], preferred_element_type=jnp.float32)
             + ba_ref[...])
  h2 = _relu(jnp.dot(h1, wb_ref[...], preferred_element_type=jnp.float32)
             + bb_ref[...])
  out_ref[0] = h2[:, :D_FEAT] + r_ref[0]
  out_ref[1] = h2[:, D_FEAT:] + r_ref[1]


def _mlp3_head_body(p_ref, r_ref, wa_ref, ba_ref, wb_ref, bb_ref,
                    wl1_ref, bl1_ref, wl2_ref, bl2_ref, y_ref):
  h0 = jnp.concatenate([p_ref[0], p_ref[1]], axis=1)
  h1 = _relu(jnp.dot(h0, wa_ref[...], preferred_element_type=jnp.float32)
             + ba_ref[...])
  h2 = _relu(jnp.dot(h1, wb_ref[...], preferred_element_type=jnp.float32)
             + bb_ref[...])
  x3 = h2 + jnp.concatenate([r_ref[0], r_ref[1]], axis=1)
  h = _relu(jnp.dot(x3, wl1_ref[...], preferred_element_type=jnp.float32)
            + bl1_ref[...])
  y_ref[...] = (jnp.dot(h, wl2_ref[...], preferred_element_type=jnp.float32)
                + bl2_ref[...])


def _split_spec():
  return pl.BlockSpec((NCORE, BM, D_FEAT), lambda i: (0, i, 0))


def _full_spec(shape):
  nd = len(shape)
  return pl.BlockSpec(shape, lambda i: (0,) * nd)


def _mlp1(p, wa, ba, wb, bb):
  return pl.pallas_call(
      _mlp1_body,
      grid=(N_NODES // BM,),
      in_specs=[_split_spec(), _full_spec(wa.shape), _full_spec(ba.shape),
                _full_spec(wb.shape), _full_spec(bb.shape)],
      out_specs=_split_spec(),
      out_shape=jax.ShapeDtypeStruct((NCORE, N_NODES, D_FEAT), jnp.float32),
  )(p, wa, ba, wb, bb)


def _mlp2(p, r, wa, ba, wb, bb):
  return pl.pallas_call(
      _mlp2_body,
      grid=(N_NODES // BM,),
      in_specs=[_split_spec(), _split_spec(), _full_spec(wa.shape),
                _full_spec(ba.shape), _full_spec(wb.shape),
                _full_spec(bb.shape)],
      out_specs=_split_spec(),
      out_shape=jax.ShapeDtypeStruct((NCORE, N_NODES, D_FEAT), jnp.float32),
  )(p, r, wa, ba, wb, bb)


def _mlp3_head(p, r, wa, ba, wb, bb, wl1, bl1, wl2, bl2):
  return pl.pallas_call(
      _mlp3_head_body,
      grid=(N_NODES // BM,),
      in_specs=[_split_spec(), _split_spec(), _full_spec(wa.shape),
                _full_spec(ba.shape), _full_spec(wb.shape),
                _full_spec(bb.shape), _full_spec(wl1.shape),
                _full_spec(bl1.shape), _full_spec(wl2.shape),
                _full_spec(bl2.shape)],
      out_specs=pl.BlockSpec((BM, 1), lambda i: (i, 0)),
      out_shape=jax.ShapeDtypeStruct((N_NODES, 1), jnp.float32),
  )(p, r, wa, ba, wb, bb, wl1, bl1, wl2, bl2)


def kernel(x, edge_index, W1a, b1a, W1b, b1b, W2a, b2a, W2b, b2b,
           W3a, b3a, W3b, b3b, Wl1, bl1, Wl2, bl2):
  ei = edge_index.astype(jnp.int32)
  npad = E_PAD - N_EDGES
  pad_src = jnp.zeros((npad,), jnp.int32)
  pad_dst = N_NODES + (jnp.arange(npad, dtype=jnp.int32) % 8)
  srcm = jnp.concatenate([ei[0], pad_src]).reshape(NSPAN, CPT, CHUNK)
  dstm = jnp.concatenate([ei[1], pad_dst]).reshape(NSPAN, CPT, CHUNK)
  zrows = jnp.zeros((RMAIN, D_FEAT), jnp.float32)
  b1a2, b1b2 = b1a.reshape(1, -1), b1b.reshape(1, -1)
  b2a2, b2b2 = b2a.reshape(1, -1), b2b.reshape(1, -1)
  b3a2, b3b2 = b3a.reshape(1, -1), b3b.reshape(1, -1)
  bl12, bl22 = bl1.reshape(1, -1), bl2.reshape(1, -1)

  p1 = _agg1(x, zrows, srcm, dstm)
  x1 = _mlp1(p1, W1a, b1a2, W1b, b1b2)
  p2 = _agg23(x1, srcm, dstm)
  x2 = _mlp2(p2, x1, W2a, b2a2, W2b, b2b2)
  p3 = _agg23(x2, srcm, dstm)
  y = _mlp3_head(p3, x2, W3a, b3a2, W3b, b3b2, Wl1, bl12, Wl2, bl22)
  return y


# speedup vs baseline: 1.0091x; 1.0091x over previous
import jax
import jax.numpy as jnp
from jax import lax
from jax.experimental import pallas as pl
from jax.experimental.pallas import tpu as pltpu
from jax.experimental.pallas import tpu_sc as plsc

N_NODES = 10000
D_FEAT = 128
HIDDEN = 256
N_EDGES = 320000

CHUNK = 128
NSUB = 16
NCORE = 2
NSPAN = NCORE * NSUB
CPT = 80
IBLK = 8
E_PAD = NSPAN * CPT * CHUNK
AGG_ROWS = N_NODES + 8
RMAIN = 624
RTAIL = N_NODES - NSUB * RMAIN

BM = 1000


def _run_spans(x_rows_hbm, srcm_hbm, dstm_hbm, agg, sbuf, dbuf, rows,
               gsem0, gsem1, ssem0, ssem1, span_ids):
  nsp = len(span_ids)
  T = nsp * CPT
  nblk = T // IBLK

  def span_chunk(g):
    if nsp == 1:
      return span_ids[0], g
    sp = g // CPT
    return jnp.where(sp == 0, span_ids[0], span_ids[1]), g - sp * CPT

  def stage(kb):
    w, gg = span_chunk(kb * IBLK)
    o = pl.multiple_of(gg, IBLK)
    d = pl.ds(pl.multiple_of(lax.rem(kb, 2) * IBLK, IBLK), IBLK)
    pltpu.sync_copy(srcm_hbm.at[w].at[pl.ds(o, IBLK)], sbuf.at[d])
    pltpu.sync_copy(dstm_hbm.at[w].at[pl.ds(o, IBLK)], dbuf.at[d])

  def gather(g, rbuf, sem):
    return pltpu.make_async_copy(
        x_rows_hbm.at[sbuf.at[lax.rem(g, 2 * IBLK)]], rbuf, sem)

  def scatter_start(g, rbuf, sem):
    pltpu.async_copy(rbuf, agg.at[dbuf.at[lax.rem(g, 2 * IBLK)]], sem,
                     add=True)

  def scatter_wait(g, rbuf, sem):
    pltpu.make_async_copy(rbuf, agg.at[dbuf.at[lax.rem(g, 2 * IBLK)]],
                          sem).wait()

  stage(0)
  gather(0, rows.at[0], gsem0).start()

  def body(t, carry):
    g0 = 2 * t
    g1 = g0 + 1

    @pl.when(lax.rem(g0, IBLK) == 0)
    def _():
      kb = g0 // IBLK

      @pl.when(kb + 1 < nblk)
      def _():
        stage(kb + 1)

    gather(g0, rows.at[0], gsem0).wait()
    gather(g1, rows.at[1], gsem1).start()
    scatter_start(g0, rows.at[0], ssem0)
    scatter_wait(g0, rows.at[0], ssem0)

    gather(g1, rows.at[1], gsem1).wait()

    @pl.when(g1 + 1 < T)
    def _():
      gather(g1 + 1, rows.at[0], gsem0).start()

    scatter_start(g1, rows.at[1], ssem1)
    scatter_wait(g1, rows.at[1], ssem1)
    return carry

  lax.fori_loop(0, T // 2, body, 0)


def _copy_node_rows(src_get, dst_put, s):
  r0 = s * RMAIN
  dst_put(pl.ds(r0, RMAIN), src_get(pl.ds(r0, RMAIN)))

  @pl.when(s == 0)
  def _():
    dst_put(pl.ds(NSUB * RMAIN, RTAIL), src_get(pl.ds(NSUB * RMAIN, RTAIL)))


def _agg1_body(x_hbm, z_hbm, srcm_hbm, dstm_hbm, out_hbm,
               agg, sbuf, dbuf, rows, gsem0, gsem1, ssem0, ssem1):
  c = lax.axis_index("c")
  s = lax.axis_index("s")

  @pl.when(c == 0)
  def _():
    _copy_node_rows(lambda d: x_hbm.at[d], lambda d, r: pltpu.sync_copy(r, agg.at[d]), s)

  @pl.when(c != 0)
  def _():
    pltpu.sync_copy(z_hbm, agg.at[pl.ds(s * RMAIN, RMAIN)])

    @pl.when(s == 0)
    def _():
      pltpu.sync_copy(z_hbm.at[pl.ds(0, RTAIL)],
                      agg.at[pl.ds(NSUB * RMAIN, RTAIL)])

  plsc.subcore_barrier()
  _run_spans(x_hbm, srcm_hbm, dstm_hbm, agg, sbuf, dbuf, rows,
             gsem0, gsem1, ssem0, ssem1, [c * NSUB + s])
  plsc.subcore_barrier()
  _copy_node_rows(lambda d: agg.at[d],
                  lambda d, r: pltpu.sync_copy(r, out_hbm.at[c].at[d]), s)


def _agg23_body(xs_hbm, srcm_hbm, dstm_hbm, out_hbm,
                agg, sbuf, dbuf, rows, gsem0, gsem1, ssem0, ssem1):
  c = lax.axis_index("c")
  s = lax.axis_index("s")
  xc = xs_hbm.at[c]
  _copy_node_rows(lambda d: xc.at[d], lambda d, r: pltpu.sync_copy(r, agg.at[d]), s)
  plsc.subcore_barrier()
  _run_spans(xc, srcm_hbm, dstm_hbm, agg, sbuf, dbuf, rows,
             gsem0, gsem1, ssem0, ssem1, [2 * s, 2 * s + 1])
  plsc.subcore_barrier()
  _copy_node_rows(lambda d: agg.at[d],
                  lambda d, r: pltpu.sync_copy(r, out_hbm.at[c].at[d]), s)


def _make_agg(body):
  mesh = plsc.VectorSubcoreMesh(core_axis_name="c", subcore_axis_name="s")
  return pl.kernel(
      body,
      out_type=jax.ShapeDtypeStruct((NCORE, N_NODES, D_FEAT), jnp.float32),
      mesh=mesh,
      scratch_types=[
          pltpu.VMEM_SHARED((AGG_ROWS, D_FEAT), jnp.float32),
          pltpu.VMEM((2 * IBLK, CHUNK), jnp.int32),
          pltpu.VMEM((2 * IBLK, CHUNK), jnp.int32),
          pltpu.VMEM((2, CHUNK, D_FEAT), jnp.float32),
          pltpu.SemaphoreType.DMA,
          pltpu.SemaphoreType.DMA,
          pltpu.SemaphoreType.DMA,
          pltpu.SemaphoreType.DMA,
      ],
  )


_agg1 = _make_agg(_agg1_body)
_agg23 = _make_agg(_agg23_body)


def _relu(v):
  return jnp.maximum(v, 0.0)


def _mlp1_body(p_ref, wa_ref, ba_ref, wb_ref, bb_ref, out_ref):
  h0 = p_ref[0] + p_ref[1]
  h1 = _relu(jnp.dot(h0, wa_ref[...], preferred_element_type=jnp.float32)
             + ba_ref[...])
  h2 = _relu(jnp.dot(h1, wb_ref[...], preferred_element_type=jnp.float32)
             + bb_ref[...])
  out_ref[0] = h2[:, :D_FEAT]
  out_ref[1] = h2[:, D_FEAT:]


def _mlp2_body(p_ref, r_ref, wa_ref, ba_ref, wb_ref, bb_ref, out_ref):
  h0 = jnp.concatenate([p_ref[0], p_ref[1]], axis=1)
  h1 = _relu(jnp.dot(h0, wa_ref[...], preferred_element_type=jnp.float32)
             + ba_ref[...])
  h2 = _relu(jnp.dot(h1, wb_ref[...], preferred_element_type=jnp.float32)
             + bb_ref[...])
  out_ref[0] = h2[:, :D_FEAT] + r_ref[0]
  out_ref[1] = h2[:, D_FEAT:] + r_ref[1]


def _mlp3_head_body(p_ref, r_ref, wa_ref, ba_ref, wb_ref, bb_ref,
                    wl1_ref, bl1_ref, wl2_ref, bl2_ref, y_ref):
  h0 = jnp.concatenate([p_ref[0], p_ref[1]], axis=1)
  h1 = _relu(jnp.dot(h0, wa_ref[...], preferred_element_type=jnp.float32)
             + ba_ref[...])
  h2 = _relu(jnp.dot(h1, wb_ref[...], preferred_element_type=jnp.float32)
             + bb_ref[...])
  x3 = h2 + jnp.concatenate([r_ref[0], r_ref[1]], axis=1)
  h = _relu(jnp.dot(x3, wl1_ref[...], preferred_element_type=jnp.float32)
            + bl1_ref[...])
  y_ref[...] = (jnp.dot(h, wl2_ref[...], preferred_element_type=jnp.float32)
                + bl2_ref[...])


def _split_spec():
  return pl.BlockSpec((NCORE, BM, D_FEAT), lambda i: (0, i, 0))


def _full_spec(shape):
  nd = len(shape)
  return pl.BlockSpec(shape, lambda i: (0,) * nd)


def _mlp1(p, wa, ba, wb, bb):
  return pl.pallas_call(
      _mlp1_body,
      grid=(N_NODES // BM,),
      in_specs=[_split_spec(), _full_spec(wa.shape), _full_spec(ba.shape),
                _full_spec(wb.shape), _full_spec(bb.shape)],
      out_specs=_split_spec(),
      out_shape=jax.ShapeDtypeStruct((NCORE, N_NODES, D_FEAT), jnp.float32),
  )(p, wa, ba, wb, bb)


def _mlp2(p, r, wa, ba, wb, bb):
  return pl.pallas_call(
      _mlp2_body,
      grid=(N_NODES // BM,),
      in_specs=[_split_spec(), _split_spec(), _full_spec(wa.shape),
                _full_spec(ba.shape), _full_spec(wb.shape),
                _full_spec(bb.shape)],
      out_specs=_split_spec(),
      out_shape=jax.ShapeDtypeStruct((NCORE, N_NODES, D_FEAT), jnp.float32),
  )(p, r, wa, ba, wb, bb)


def _mlp3_head(p, r, wa, ba, wb, bb, wl1, bl1, wl2, bl2):
  return pl.pallas_call(
      _mlp3_head_body,
      grid=(N_NODES // BM,),
      in_specs=[_split_spec(), _split_spec(), _full_spec(wa.shape),
                _full_spec(ba.shape), _full_spec(wb.shape),
                _full_spec(bb.shape), _full_spec(wl1.shape),
                _full_spec(bl1.shape), _full_spec(wl2.shape),
                _full_spec(bl2.shape)],
      out_specs=pl.BlockSpec((BM, 1), lambda i: (i, 0)),
      out_shape=jax.ShapeDtypeStruct((N_NODES, 1), jnp.float32),
  )(p, r, wa, ba, wb, bb, wl1, bl1, wl2, bl2)


def kernel(x, edge_index, W1a, b1a, W1b, b1b, W2a, b2a, W2b, b2b,
           W3a, b3a, W3b, b3b, Wl1, bl1, Wl2, bl2):
  ei = edge_index.astype(jnp.int32)
  npad = E_PAD - N_EDGES
  pad_src = jnp.zeros((npad,), jnp.int32)
  pad_dst = N_NODES + (jnp.arange(npad, dtype=jnp.int32) % 8)
  srcm = jnp.concatenate([ei[0], pad_src]).reshape(NSPAN, CPT, CHUNK)
  dstm = jnp.concatenate([ei[1], pad_dst]).reshape(NSPAN, CPT, CHUNK)
  zrows = jnp.zeros((RMAIN, D_FEAT), jnp.float32)
  b1a2, b1b2 = b1a.reshape(1, -1), b1b.reshape(1, -1)
  b2a2, b2b2 = b2a.reshape(1, -1), b2b.reshape(1, -1)
  b3a2, b3b2 = b3a.reshape(1, -1), b3b.reshape(1, -1)
  bl12, bl22 = bl1.reshape(1, -1), bl2.reshape(1, -1)

  p1 = _agg1(x, zrows, srcm, dstm)
  x1 = _mlp1(p1, W1a, b1a2, W1b, b1b2)
  p2 = _agg23(x1, srcm, dstm)
  x2 = _mlp2(p2, x1, W2a, b2a2, W2b, b2b2)
  p3 = _agg23(x2, srcm, dstm)
  y = _mlp3_head(p3, x2, W3a, b3a2, W3b, b3b2, Wl1, bl12, Wl2, bl22)
  return y


# speedup vs baseline: 1.0136x; 1.0045x over previous
import jax
import jax.numpy as jnp
from jax import lax
from jax.experimental import pallas as pl
from jax.experimental.pallas import tpu as pltpu
from jax.experimental.pallas import tpu_sc as plsc

N_NODES = 10000
D_FEAT = 128
HIDDEN = 256
N_EDGES = 320000

CHUNK = 128
NSUB = 16
NCORE = 2
NSPAN = NCORE * NSUB
CPT = 80
IBLK = 8
E_PAD = NSPAN * CPT * CHUNK
AGG_ROWS = N_NODES + 8
RMAIN = 624
RTAIL = N_NODES - NSUB * RMAIN

BM = 1000


def _run_spans(x_rows_hbm, srcm_hbm, dstm_hbm, agg, sbuf, dbuf, rows,
               gsem0, gsem1, ssem0, ssem1, isem, span_ids):
  nsp = len(span_ids)
  T = nsp * CPT
  nblk = T // IBLK

  def span_chunk(g):
    if nsp == 1:
      return span_ids[0], g
    sp = g // CPT
    return jnp.where(sp == 0, span_ids[0], span_ids[1]), g - sp * CPT

  def _stage_copies(kb):
    w, gg = span_chunk(kb * IBLK)
    o = pl.multiple_of(gg, IBLK)
    d = pl.ds(pl.multiple_of(lax.rem(kb, 2) * IBLK, IBLK), IBLK)
    return (pltpu.make_async_copy(srcm_hbm.at[w].at[pl.ds(o, IBLK)],
                                  sbuf.at[d], isem),
            pltpu.make_async_copy(dstm_hbm.at[w].at[pl.ds(o, IBLK)],
                                  dbuf.at[d], isem))

  def stage_start(kb):
    for cp in _stage_copies(kb):
      cp.start()

  def stage_wait(kb):
    for cp in _stage_copies(kb):
      cp.wait()

  def gather(g, rbuf, sem):
    return pltpu.make_async_copy(
        x_rows_hbm.at[sbuf.at[lax.rem(g, 2 * IBLK)]], rbuf, sem)

  def scatter_start(g, rbuf, sem):
    pltpu.async_copy(rbuf, agg.at[dbuf.at[lax.rem(g, 2 * IBLK)]], sem,
                     add=True)

  def scatter_wait(g, rbuf, sem):
    pltpu.make_async_copy(rbuf, agg.at[dbuf.at[lax.rem(g, 2 * IBLK)]],
                          sem).wait()

  stage_start(0)
  stage_wait(0)
  gather(0, rows.at[0], gsem0).start()

  def body(t, carry):
    g0 = 2 * t
    g1 = g0 + 1

    @pl.when(lax.rem(g0, IBLK) == 0)
    def _():
      kb = g0 // IBLK

      @pl.when(kb + 1 < nblk)
      def _():
        stage_start(kb + 1)

    gather(g0, rows.at[0], gsem0).wait()
    gather(g1, rows.at[1], gsem1).start()
    scatter_start(g0, rows.at[0], ssem0)
    scatter_wait(g0, rows.at[0], ssem0)

    gather(g1, rows.at[1], gsem1).wait()

    @pl.when(g1 + 1 < T)
    def _():
      nxt = g1 + 1

      @pl.when(lax.rem(nxt, IBLK) == 0)
      def _():
        stage_wait(nxt // IBLK)

      gather(nxt, rows.at[0], gsem0).start()

    scatter_start(g1, rows.at[1], ssem1)
    scatter_wait(g1, rows.at[1], ssem1)
    return carry

  lax.fori_loop(0, T // 2, body, 0)


def _copy_node_rows(src_get, dst_put, s):
  r0 = s * RMAIN
  dst_put(pl.ds(r0, RMAIN), src_get(pl.ds(r0, RMAIN)))

  @pl.when(s == 0)
  def _():
    dst_put(pl.ds(NSUB * RMAIN, RTAIL), src_get(pl.ds(NSUB * RMAIN, RTAIL)))


def _agg1_body(x_hbm, z_hbm, srcm_hbm, dstm_hbm, out_hbm,
               agg, sbuf, dbuf, rows, gsem0, gsem1, ssem0, ssem1, isem):
  c = lax.axis_index("c")
  s = lax.axis_index("s")

  @pl.when(c == 0)
  def _():
    _copy_node_rows(lambda d: x_hbm.at[d], lambda d, r: pltpu.sync_copy(r, agg.at[d]), s)

  @pl.when(c != 0)
  def _():
    pltpu.sync_copy(z_hbm, agg.at[pl.ds(s * RMAIN, RMAIN)])

    @pl.when(s == 0)
    def _():
      pltpu.sync_copy(z_hbm.at[pl.ds(0, RTAIL)],
                      agg.at[pl.ds(NSUB * RMAIN, RTAIL)])

  plsc.subcore_barrier()
  _run_spans(x_hbm, srcm_hbm, dstm_hbm, agg, sbuf, dbuf, rows,
             gsem0, gsem1, ssem0, ssem1, isem, [c * NSUB + s])
  plsc.subcore_barrier()
  _copy_node_rows(lambda d: agg.at[d],
                  lambda d, r: pltpu.sync_copy(r, out_hbm.at[c].at[d]), s)


def _agg23_body(xs_hbm, srcm_hbm, dstm_hbm, out_hbm,
                agg, sbuf, dbuf, rows, gsem0, gsem1, ssem0, ssem1, isem):
  c = lax.axis_index("c")
  s = lax.axis_index("s")
  xc = xs_hbm.at[c]
  _copy_node_rows(lambda d: xc.at[d], lambda d, r: pltpu.sync_copy(r, agg.at[d]), s)
  plsc.subcore_barrier()
  _run_spans(xc, srcm_hbm, dstm_hbm, agg, sbuf, dbuf, rows,
             gsem0, gsem1, ssem0, ssem1, isem, [2 * s, 2 * s + 1])
  plsc.subcore_barrier()
  _copy_node_rows(lambda d: agg.at[d],
                  lambda d, r: pltpu.sync_copy(r, out_hbm.at[c].at[d]), s)


def _make_agg(body):
  mesh = plsc.VectorSubcoreMesh(core_axis_name="c", subcore_axis_name="s")
  return pl.kernel(
      body,
      out_type=jax.ShapeDtypeStruct((NCORE, N_NODES, D_FEAT), jnp.float32),
      mesh=mesh,
      scratch_types=[
          pltpu.VMEM_SHARED((AGG_ROWS, D_FEAT), jnp.float32),
          pltpu.VMEM((2 * IBLK, CHUNK), jnp.int32),
          pltpu.VMEM((2 * IBLK, CHUNK), jnp.int32),
          pltpu.VMEM((2, CHUNK, D_FEAT), jnp.float32),
          pltpu.SemaphoreType.DMA,
          pltpu.SemaphoreType.DMA,
          pltpu.SemaphoreType.DMA,
          pltpu.SemaphoreType.DMA,
          pltpu.SemaphoreType.DMA,
      ],
  )


_agg1 = _make_agg(_agg1_body)
_agg23 = _make_agg(_agg23_body)


def _relu(v):
  return jnp.maximum(v, 0.0)


def _mlp1_body(p_ref, wa_ref, ba_ref, wb_ref, bb_ref, out_ref):
  h0 = p_ref[0] + p_ref[1]
  h1 = _relu(jnp.dot(h0, wa_ref[...], preferred_element_type=jnp.float32)
             + ba_ref[...])
  h2 = _relu(jnp.dot(h1, wb_ref[...], preferred_element_type=jnp.float32)
             + bb_ref[...])
  out_ref[0] = h2[:, :D_FEAT]
  out_ref[1] = h2[:, D_FEAT:]


def _mlp2_body(p_ref, r_ref, wa_ref, ba_ref, wb_ref, bb_ref, out_ref):
  h0 = jnp.concatenate([p_ref[0], p_ref[1]], axis=1)
  h1 = _relu(jnp.dot(h0, wa_ref[...], preferred_element_type=jnp.float32)
             + ba_ref[...])
  h2 = _relu(jnp.dot(h1, wb_ref[...], preferred_element_type=jnp.float32)
             + bb_ref[...])
  out_ref[0] = h2[:, :D_FEAT] + r_ref[0]
  out_ref[1] = h2[:, D_FEAT:] + r_ref[1]


def _mlp3_head_body(p_ref, r_ref, wa_ref, ba_ref, wb_ref, bb_ref,
                    wl1_ref, bl1_ref, wl2_ref, bl2_ref, y_ref):
  h0 = jnp.concatenate([p_ref[0], p_ref[1]], axis=1)
  h1 = _relu(jnp.dot(h0, wa_ref[...], preferred_element_type=jnp.float32)
             + ba_ref[...])
  h2 = _relu(jnp.dot(h1, wb_ref[...], preferred_element_type=jnp.float32)
             + bb_ref[...])
  x3 = h2 + jnp.concatenate([r_ref[0], r_ref[1]], axis=1)
  h = _relu(jnp.dot(x3, wl1_ref[...], preferred_element_type=jnp.float32)
            + bl1_ref[...])
  y_ref[...] = (jnp.dot(h, wl2_ref[...], preferred_element_type=jnp.float32)
                + bl2_ref[...])


def _split_spec():
  return pl.BlockSpec((NCORE, BM, D_FEAT), lambda i: (0, i, 0))


def _full_spec(shape):
  nd = len(shape)
  return pl.BlockSpec(shape, lambda i: (0,) * nd)


def _mlp1(p, wa, ba, wb, bb):
  return pl.pallas_call(
      _mlp1_body,
      grid=(N_NODES // BM,),
      in_specs=[_split_spec(), _full_spec(wa.shape), _full_spec(ba.shape),
                _full_spec(wb.shape), _full_spec(bb.shape)],
      out_specs=_split_spec(),
      out_shape=jax.ShapeDtypeStruct((NCORE, N_NODES, D_FEAT), jnp.float32),
  )(p, wa, ba, wb, bb)


def _mlp2(p, r, wa, ba, wb, bb):
  return pl.pallas_call(
      _mlp2_body,
      grid=(N_NODES // BM,),
      in_specs=[_split_spec(), _split_spec(), _full_spec(wa.shape),
                _full_spec(ba.shape), _full_spec(wb.shape),
                _full_spec(bb.shape)],
      out_specs=_split_spec(),
      out_shape=jax.ShapeDtypeStruct((NCORE, N_NODES, D_FEAT), jnp.float32),
  )(p, r, wa, ba, wb, bb)


def _mlp3_head(p, r, wa, ba, wb, bb, wl1, bl1, wl2, bl2):
  return pl.pallas_call(
      _mlp3_head_body,
      grid=(N_NODES // BM,),
      in_specs=[_split_spec(), _split_spec(), _full_spec(wa.shape),
                _full_spec(ba.shape), _full_spec(wb.shape),
                _full_spec(bb.shape), _full_spec(wl1.shape),
                _full_spec(bl1.shape), _full_spec(wl2.shape),
                _full_spec(bl2.shape)],
      out_specs=pl.BlockSpec((BM, 1), lambda i: (i, 0)),
      out_shape=jax.ShapeDtypeStruct((N_NODES, 1), jnp.float32),
  )(p, r, wa, ba, wb, bb, wl1, bl1, wl2, bl2)


def kernel(x, edge_index, W1a, b1a, W1b, b1b, W2a, b2a, W2b, b2b,
           W3a, b3a, W3b, b3b, Wl1, bl1, Wl2, bl2):
  ei = edge_index.astype(jnp.int32)
  npad = E_PAD - N_EDGES
  pad_src = jnp.zeros((npad,), jnp.int32)
  pad_dst = N_NODES + (jnp.arange(npad, dtype=jnp.int32) % 8)
  srcm = jnp.concatenate([ei[0], pad_src]).reshape(NSPAN, CPT, CHUNK)
  dstm = jnp.concatenate([ei[1], pad_dst]).reshape(NSPAN, CPT, CHUNK)
  zrows = jnp.zeros((RMAIN, D_FEAT), jnp.float32)
  b1a2, b1b2 = b1a.reshape(1, -1), b1b.reshape(1, -1)
  b2a2, b2b2 = b2a.reshape(1, -1), b2b.reshape(1, -1)
  b3a2, b3b2 = b3a.reshape(1, -1), b3b.reshape(1, -1)
  bl12, bl22 = bl1.reshape(1, -1), bl2.reshape(1, -1)

  p1 = _agg1(x, zrows, srcm, dstm)
  x1 = _mlp1(p1, W1a, b1a2, W1b, b1b2)
  p2 = _agg23(x1, srcm, dstm)
  x2 = _mlp2(p2, x1, W2a, b2a2, W2b, b2b2)
  p3 = _agg23(x2, srcm, dstm)
  y = _mlp3_head(p3, x2, W3a, b3a2, W3b, b3b2, Wl1, bl12, Wl2, bl22)
  return y
